# Initial kernel scaffold; baseline (speedup 1.0000x reference)
import jax
import jax.numpy as jnp
from jax.experimental import pallas as pl


def kernel(pos_onehot, w1, b1, g1, be1, a1, w2, b2, g2, be2, a2):
    raise NotImplementedError("write your pallas kernel here")


# speedup vs baseline: 1.6358x; 1.6358x over previous
import numpy as np
import jax
import jax.numpy as jnp
from jax.experimental import pallas as pl
from jax.experimental.pallas import tpu as pltpu

EPS = 1e-5


def _stream_kernel(x_ref, w_ref, idx_ref, hist_ref):
    t = pl.program_id(1)

    @pl.when(t == 0)
    def _init():
        hist_ref[...] = jnp.zeros_like(hist_ref)

    x = x_ref[...]
    hist_ref[...] += jnp.sum(x, axis=0, keepdims=True)[None]

    d = jnp.dot(x.astype(jnp.bfloat16), w_ref[...],
                preferred_element_type=jnp.float32)
    idx_ref[...] = jnp.sum(d, axis=1, keepdims=True)


def _table_kernel(hist_ref, tw_ref, w1_ref, b1_ref, g1_ref, be1_ref,
                  w2_ref, b2_ref, g2_ref, be2_ref, a1_ref, a2_ref,
                  table_ref):
    classes = w1_ref.shape[0]
    n_rows = jnp.sum(hist_ref[...])
    inv_n = 1.0 / n_rows

    cnt_row = jnp.sum(hist_ref[:, 0, :], axis=0, keepdims=True)
    hi = jnp.floor(cnt_row * (1.0 / 64.0))
    lo = cnt_row - 64.0 * hi
    stacked = jnp.concatenate([hi, lo], axis=0)
    cnt_full = jax.lax.dot_general(
        stacked.astype(jnp.bfloat16), tw_ref[...],
        (((0,), (0,)), ((), ())),
        preferred_element_type=jnp.float32)
    cnt = cnt_full[:, 0:1]

    a1 = a1_ref[0, 0]
    a2 = a2_ref[0, 0]

    h = w1_ref[...] + b1_ref[...]
    mean1 = jnp.sum(h * cnt, axis=0, keepdims=True) * inv_n
    d = h - mean1
    var1 = jnp.sum(d * d * cnt, axis=0, keepdims=True) * inv_n
    scale1 = jax.lax.rsqrt(var1 + EPS) * g1_ref[...]
    z = d * scale1 + be1_ref[...]
    z = jnp.where(z > 0, z, a1 * z)

    y = jnp.dot(z, w2_ref[...],
                preferred_element_type=jnp.float32) + b2_ref[...]
    mean2 = jnp.sum(y * cnt, axis=0, keepdims=True) * inv_n
    e = y - mean2
    var2 = jnp.sum(e * e * cnt, axis=0, keepdims=True) * inv_n
    scale2 = jax.lax.rsqrt(var2 + EPS) * g2_ref[...]
    u = e * scale2 + be2_ref[...]
    table_ref[...] = jnp.where(u > 0, u, a2 * u)


def _lookup_kernel(idx_ref, table_ref, o_ref):
    rows, classes = o_ref.shape[0], table_ref.shape[0]
    iv = idx_ref[...].astype(jnp.int32)
    lane = jax.lax.broadcasted_iota(jnp.int32, (rows, classes), 1)
    onehot = (lane == iv).astype(jnp.bfloat16)
    o_ref[...] = jnp.dot(onehot, table_ref[...].astype(jnp.bfloat16),
                         preferred_element_type=jnp.float32)


def kernel(pos_onehot, w1, b1, g1, be1, a1, w2, b2, g2, be2, a2):
    b, l, classes = pos_onehot.shape
    hid = w1.shape[1]
    out_dim = w2.shape[1]
    n = b * l

    x = pos_onehot.reshape(n, classes)

    tiles = 8
    rows = n // tiles
    t_inner = tiles // 2

    cgrid = np.arange(classes)
    wnp = np.zeros((classes, 128), np.float32)
    wnp[:, 0] = (cgrid // 128) * 128
    wnp[:, 1] = cgrid % 128
    w_idx = jnp.asarray(wnp, dtype=jnp.bfloat16)

    const = lambda shape: pl.BlockSpec(shape, lambda c, t, s=len(shape): (0,) * s)

    idx, hist = pl.pallas_call(
        _stream_kernel,
        out_shape=(jax.ShapeDtypeStruct((n, 1), jnp.float32),
                   jax.ShapeDtypeStruct((2, 1, classes), jnp.float32)),
        grid=(2, t_inner),
        in_specs=[
            pl.BlockSpec((rows, classes), lambda c, t: (c * t_inner + t, 0)),
            const((classes, 128)),
        ],
        out_specs=(pl.BlockSpec((rows, 1), lambda c, t: (c * t_inner + t, 0)),
                   pl.BlockSpec((1, 1, classes), lambda c, t: (c, 0, 0))),
        compiler_params=pltpu.CompilerParams(
            dimension_semantics=("parallel", "arbitrary")),
    )(x, w_idx)

    twnp = np.zeros((2, 128), np.float32)
    twnp[0, :] = 64.0
    twnp[1, :] = 1.0
    t_w = jnp.asarray(twnp, dtype=jnp.bfloat16)

    whole = lambda arr: pl.BlockSpec(arr.shape, lambda *a: (0,) * arr.ndim)
    smem = pl.BlockSpec(memory_space=pltpu.MemorySpace.SMEM)

    table = pl.pallas_call(
        _table_kernel,
        out_shape=jax.ShapeDtypeStruct((classes, out_dim), jnp.float32),
        in_specs=[
            whole(hist), whole(t_w), whole(w1), whole(b1), whole(g1),
            whole(be1), whole(w2), whole(b2), whole(g2), whole(be2),
            smem, smem,
        ],
        out_specs=pl.BlockSpec((classes, out_dim), lambda: (0, 0)),
    )(hist, t_w, w1, b1, g1, be1, w2, b2, g2, be2, a1, a2)

    out = pl.pallas_call(
        _lookup_kernel,
        out_shape=jax.ShapeDtypeStruct((n, out_dim), jnp.float32),
        grid=(2, t_inner),
        in_specs=[
            pl.BlockSpec((rows, 1), lambda c, t: (c * t_inner + t, 0)),
            const((classes, out_dim)),
        ],
        out_specs=pl.BlockSpec((rows, out_dim), lambda c, t: (c * t_inner + t, 0)),
        compiler_params=pltpu.CompilerParams(
            dimension_semantics=("parallel", "arbitrary")),
    )(idx, table)

    return out.reshape(b, l, out_dim)
